# Initial kernel scaffold; baseline (speedup 1.0000x reference)
import jax
import jax.numpy as jnp
from jax.experimental import pallas as pl


def kernel(x, dc):
    raise NotImplementedError("write your pallas kernel here")


# speedup vs baseline: 62.7412x; 62.7412x over previous
import functools

import jax
import jax.numpy as jnp
from jax import lax
from jax.experimental import pallas as pl
from jax.experimental.pallas import tpu as pltpu
from jax.experimental.pallas import tpu_sc as plsc

_L = 16


def _shuf(v, idx):
    return v.at[idx].get(mode="promise_in_bounds")


def _bfly(v, lanes, op):
    for s in (1, 2, 4, 8):
        v = op(v, _shuf(v, lanes ^ s))
    return v


def _sc_body(nrows, p, rpw, nc, x_hbm, dc_hbm, out_hbm, rowbuf, outbuf, dcbuf):
    nvec = p // _L
    wid = lax.axis_index("s") * nc + lax.axis_index("c")
    lanes = lax.broadcasted_iota(jnp.int32, (_L,), 0)
    neg_inf = jnp.full((_L,), -jnp.inf, jnp.float32)

    pltpu.sync_copy(dc_hbm, dcbuf)
    dcv = dcbuf[...]
    pw = jnp.ones((_L,), jnp.float32)
    for k in range(_L - 1):
        pw = jnp.where(lanes > k, pw * dcv, pw)
    cw = 1.0 / (1.0 + jnp.exp(pw))
    denom = jnp.float32(p) - _bfly(cw, lanes, jnp.add)

    def row_step(i, yvs):
        r = wid * rpw + i
        pltpu.sync_copy(x_hbm.at[r], rowbuf)

        def body(k, carry):
            acc, a0, a1, a2, a3 = carry
            v = rowbuf[pl.ds(k * _L, _L)]
            acc = acc + v
            b = jnp.minimum(a0, v)
            a0 = jnp.maximum(a0, v)
            b2 = jnp.minimum(a1, b)
            a1 = jnp.maximum(a1, b)
            b3 = jnp.minimum(a2, b2)
            a2 = jnp.maximum(a2, b2)
            a3 = jnp.maximum(a3, b3)
            return acc, a0, a1, a2, a3

        acc, a0, a1, a2, a3 = lax.fori_loop(
            0, nvec, body,
            (jnp.zeros((_L,), jnp.float32), neg_inf, neg_inf, neg_inf, neg_inf))
        total = _bfly(acc, lanes, jnp.add)

        gvec = jnp.zeros((_L,), jnp.float32)
        for k in range(4):
            g = _bfly(a0, lanes, jnp.maximum)
            first = _bfly(jnp.where(a0 == g, lanes, _L), lanes, jnp.minimum)
            hit = lanes == first
            a0 = jnp.where(hit, a1, a0)
            a1 = jnp.where(hit, a2, a1)
            a2 = jnp.where(hit, a3, a2)
            a3 = jnp.where(hit, neg_inf, a3)
            gvec = jnp.where(lanes == k, g, gvec)

        y = (total - _bfly(cw * gvec, lanes, jnp.add)) / denom
        yv0, yv1 = yvs
        yv0 = jnp.where(lanes == i, y, yv0)
        yv1 = jnp.where(lanes == i - _L, y, yv1)
        return yv0, yv1

    zero = jnp.zeros((_L,), jnp.float32)
    yv0, yv1 = lax.fori_loop(0, rpw, row_step, (zero, zero))
    outbuf[pl.ds(0, _L)] = yv0
    outbuf[pl.ds(_L, _L)] = yv1
    pltpu.sync_copy(outbuf.at[pl.ds(0, rpw)], out_hbm.at[pl.ds(wid * rpw, rpw)])


def kernel(x, dc):
    b, c, h, w = x.shape
    nrows, p = b * c, h * w
    info = plsc.get_sparse_core_info()
    nc, ns = info.num_cores, info.num_subcores
    nw = nc * ns
    assert nrows % nw == 0 and p % _L == 0
    rpw = nrows // nw

    xr = x.reshape(nrows, p)
    dc16 = jnp.broadcast_to(dc.astype(jnp.float32), (_L,))

    mesh = plsc.VectorSubcoreMesh(core_axis_name="c", subcore_axis_name="s")
    kern = functools.partial(
        pl.kernel,
        mesh=mesh,
        out_type=jax.ShapeDtypeStruct((nrows,), jnp.float32),
        scratch_types=[
            pltpu.VMEM((p,), jnp.float32),
            pltpu.VMEM((2 * _L,), jnp.float32),
            pltpu.VMEM((_L,), jnp.float32),
        ],
    )(functools.partial(_sc_body, nrows, p, rpw, nc))
    return kern(xr, dc16).reshape(b, c)


# speedup vs baseline: 107.4112x; 1.7120x over previous
import functools

import jax
import jax.numpy as jnp
from jax import lax
from jax.experimental import pallas as pl
from jax.experimental.pallas import tpu as pltpu
from jax.experimental.pallas import tpu_sc as plsc

_L = 16


def _shuf(v, idx):
    return v.at[idx].get(mode="promise_in_bounds")


def _bfly(v, lanes, op):
    for s in (1, 2, 4, 8):
        v = op(v, _shuf(v, lanes ^ s))
    return v


def _insert(st, v):
    a0, a1, a2, a3 = st
    b = jnp.minimum(a0, v)
    a0 = jnp.maximum(a0, v)
    b2 = jnp.minimum(a1, b)
    a1 = jnp.maximum(a1, b)
    b3 = jnp.minimum(a2, b2)
    a2 = jnp.maximum(a2, b2)
    a3 = jnp.maximum(a3, b3)
    return (a0, a1, a2, a3)


_U = 4


def _sc_body(nrows, p, rpw, nc, x_hbm, dc_hbm, out_hbm, buf0, buf1, outbuf,
             dcbuf, sem0, sem1):
    nvec = p // _L
    wid = lax.axis_index("s") * nc + lax.axis_index("c")
    lanes = lax.broadcasted_iota(jnp.int32, (_L,), 0)
    neg_inf = jnp.full((_L,), -jnp.inf, jnp.float32)

    pltpu.sync_copy(dc_hbm, dcbuf)
    dcv = dcbuf[...]
    pw = jnp.ones((_L,), jnp.float32)
    for k in range(_L - 1):
        pw = jnp.where(lanes > k, pw * dcv, pw)
    cw = 1.0 / (1.0 + jnp.exp(pw))
    denom = jnp.float32(p) - _bfly(cw, lanes, jnp.add)

    zero = jnp.zeros((_L,), jnp.float32)
    base = wid * rpw

    def row_compute(buf):
        def body(k, carry):
            accs, sts = carry
            o = k * (_U * _L)
            vs = [buf[pl.ds(o + u * _L, _L)] for u in range(_U)]
            accs = tuple(a + v for a, v in zip(accs, vs))
            sts = tuple(_insert(st, v) for st, v in zip(sts, vs))
            return accs, sts

        init = (
            (zero,) * _U,
            tuple((neg_inf, neg_inf, neg_inf, neg_inf) for _ in range(_U)),
        )
        accs, sts = lax.fori_loop(0, nvec // _U, body, init)

        acc = accs[0] + accs[1] + (accs[2] + accs[3])
        total = _bfly(acc, lanes, jnp.add)
        a0, a1, a2, a3 = sts[0]
        for st in sts[1:]:
            for v in st:
                a0, a1, a2, a3 = _insert((a0, a1, a2, a3), v)

        gvec = jnp.zeros((_L,), jnp.float32)
        for k in range(4):
            g = _bfly(a0, lanes, jnp.maximum)
            first = _bfly(jnp.where(a0 == g, lanes, _L), lanes, jnp.minimum)
            hit = lanes == first
            a0 = jnp.where(hit, a1, a0)
            a1 = jnp.where(hit, a2, a1)
            a2 = jnp.where(hit, a3, a2)
            a3 = jnp.where(hit, neg_inf, a3)
            gvec = jnp.where(lanes == k, g, gvec)

        return (total - _bfly(cw * gvec, lanes, jnp.add)) / denom

    pltpu.async_copy(x_hbm.at[base], buf0, sem0)

    def pair_step(ip, yvs):
        r = base + 2 * ip
        pltpu.async_copy(x_hbm.at[r + 1], buf1, sem1)
        pltpu.make_async_copy(x_hbm.at[r], buf0, sem0).wait()
        ya = row_compute(buf0)
        rn = jnp.minimum(r + 2, nrows - 1)
        pltpu.async_copy(x_hbm.at[rn], buf0, sem0)
        pltpu.make_async_copy(x_hbm.at[r + 1], buf1, sem1).wait()
        yb = row_compute(buf1)
        yv0, yv1 = yvs
        i = 2 * ip
        yv0 = jnp.where(lanes == i, ya, yv0)
        yv0 = jnp.where(lanes == i + 1, yb, yv0)
        yv1 = jnp.where(lanes == i - _L, ya, yv1)
        yv1 = jnp.where(lanes == i + 1 - _L, yb, yv1)
        return yv0, yv1

    yv0, yv1 = lax.fori_loop(0, rpw // 2, pair_step, (zero, zero))
    pltpu.make_async_copy(x_hbm.at[base], buf0, sem0).wait()
    outbuf[pl.ds(0, _L)] = yv0
    outbuf[pl.ds(_L, _L)] = yv1
    pltpu.sync_copy(outbuf.at[pl.ds(0, rpw)], out_hbm.at[pl.ds(wid * rpw, rpw)])


def kernel(x, dc):
    b, c, h, w = x.shape
    nrows, p = b * c, h * w
    info = plsc.get_sparse_core_info()
    nc, ns = info.num_cores, info.num_subcores
    nw = nc * ns
    assert nrows % nw == 0 and p % (_L * _U) == 0
    rpw = nrows // nw

    xr = x.reshape(nrows, p)
    dc16 = jnp.broadcast_to(dc.astype(jnp.float32), (_L,))

    mesh = plsc.VectorSubcoreMesh(core_axis_name="c", subcore_axis_name="s")
    kern = functools.partial(
        pl.kernel,
        mesh=mesh,
        out_type=jax.ShapeDtypeStruct((nrows,), jnp.float32),
        scratch_types=[
            pltpu.VMEM((p,), jnp.float32),
            pltpu.VMEM((p,), jnp.float32),
            pltpu.VMEM((2 * _L,), jnp.float32),
            pltpu.VMEM((_L,), jnp.float32),
            pltpu.SemaphoreType.DMA,
            pltpu.SemaphoreType.DMA,
        ],
    )(functools.partial(_sc_body, nrows, p, rpw, nc))
    return kern(xr, dc16).reshape(b, c)


# speedup vs baseline: 109.2440x; 1.0171x over previous
import functools

import jax
import jax.numpy as jnp
from jax import lax
from jax.experimental import pallas as pl
from jax.experimental.pallas import tpu as pltpu
from jax.experimental.pallas import tpu_sc as plsc

_L = 16


def _shuf(v, idx):
    return v.at[idx].get(mode="promise_in_bounds")


def _bfly(v, lanes, op):
    for s in (1, 2, 4, 8):
        v = op(v, _shuf(v, lanes ^ s))
    return v


def _insert(st, v):
    a0, a1, a2, a3 = st
    b = jnp.minimum(a0, v)
    a0 = jnp.maximum(a0, v)
    b2 = jnp.minimum(a1, b)
    a1 = jnp.maximum(a1, b)
    b3 = jnp.minimum(a2, b2)
    a2 = jnp.maximum(a2, b2)
    a3 = jnp.maximum(a3, b3)
    return (a0, a1, a2, a3)


_U = 4


def _sc_body(nrows, p, rpw, nc, x_hbm, dc_hbm, out_hbm, buf0, buf1, outbuf,
             dcbuf, sem0, sem1):
    nvec = p // _L
    wid = lax.axis_index("s") * nc + lax.axis_index("c")
    lanes = lax.broadcasted_iota(jnp.int32, (_L,), 0)
    neg_inf = jnp.full((_L,), -jnp.inf, jnp.float32)

    pltpu.sync_copy(dc_hbm, dcbuf)
    dcv = dcbuf[...]
    pw = jnp.ones((_L,), jnp.float32)
    for k in range(_L - 1):
        pw = jnp.where(lanes > k, pw * dcv, pw)
    cw = 1.0 / (1.0 + jnp.exp(pw))
    denom = jnp.float32(p) - _bfly(cw, lanes, jnp.add)

    zero = jnp.zeros((_L,), jnp.float32)
    base = wid * rpw

    def row_compute(buf):
        def body(k, carry):
            accs, sts = carry
            o = k * (_U * _L)
            vs = [buf[pl.ds(o + u * _L, _L)] for u in range(_U)]
            accs = tuple(a + v for a, v in zip(accs, vs))
            sts = tuple(_insert(st, v) for st, v in zip(sts, vs))
            return accs, sts

        init = (
            (zero,) * _U,
            tuple((neg_inf, neg_inf, neg_inf, neg_inf) for _ in range(_U)),
        )
        accs, sts = lax.fori_loop(0, nvec // _U, body, init)

        acc = accs[0] + accs[1] + (accs[2] + accs[3])
        total = _bfly(acc, lanes, jnp.add)
        a0, a1, a2, a3 = sts[0]
        for st in sts[1:]:
            for v in st:
                a0, a1, a2, a3 = _insert((a0, a1, a2, a3), v)

        gvec = jnp.zeros((_L,), jnp.float32)
        for k in range(4):
            g = _bfly(a0, lanes, jnp.maximum)
            first = _bfly(jnp.where(a0 == g, lanes, _L), lanes, jnp.minimum)
            hit = lanes == first
            a0 = jnp.where(hit, a1, a0)
            a1 = jnp.where(hit, a2, a1)
            a2 = jnp.where(hit, a3, a2)
            a3 = jnp.where(hit, neg_inf, a3)
            gvec = jnp.where(lanes == k, g, gvec)

        return (total - _bfly(cw * gvec, lanes, jnp.add)) / denom

    pltpu.async_copy(x_hbm.at[base], buf0, sem0)

    def pair_step(ip, yvs):
        r = base + 2 * ip
        pltpu.async_copy(x_hbm.at[r + 1], buf1, sem1)
        pltpu.make_async_copy(x_hbm.at[r], buf0, sem0).wait()
        ya = row_compute(buf0)
        rn = jnp.minimum(r + 2, nrows - 1)
        pltpu.async_copy(x_hbm.at[rn], buf0, sem0)
        pltpu.make_async_copy(x_hbm.at[r + 1], buf1, sem1).wait()
        yb = row_compute(buf1)
        yv0, yv1 = yvs
        i = 2 * ip
        yv0 = jnp.where(lanes == i, ya, yv0)
        yv0 = jnp.where(lanes == i + 1, yb, yv0)
        yv1 = jnp.where(lanes == i - _L, ya, yv1)
        yv1 = jnp.where(lanes == i + 1 - _L, yb, yv1)
        return yv0, yv1

    yv0, yv1 = lax.fori_loop(0, rpw // 2, pair_step, (zero, zero))
    pltpu.make_async_copy(x_hbm.at[base], buf0, sem0).wait()
    outbuf[pl.ds(0, _L)] = yv0
    outbuf[pl.ds(_L, _L)] = yv1
    pltpu.sync_copy(outbuf.at[pl.ds(0, rpw)], out_hbm.at[pl.ds(wid * rpw, rpw)])


def _tc_body(dc_ref, x_ref, o_ref):
    p = x_ref.shape[1]
    nchunk = p // 128
    zero = jnp.zeros((8, 128), jnp.float32)
    neg_inf = jnp.full((8, 128), -jnp.inf, jnp.float32)

    def body(k, carry):
        accs, sts = carry
        vs = [x_ref[:, pl.ds((k * _U + u) * 128, 128)] for u in range(_U)]
        accs = tuple(a + v for a, v in zip(accs, vs))
        sts = tuple(_insert(st, v) for st, v in zip(sts, vs))
        return accs, sts

    init = (
        (zero,) * _U,
        tuple((neg_inf, neg_inf, neg_inf, neg_inf) for _ in range(_U)),
    )
    accs, sts = lax.fori_loop(0, nchunk // _U, body, init)

    acc = accs[0] + accs[1] + (accs[2] + accs[3])
    total = jnp.sum(acc, axis=1, keepdims=True)
    a0, a1, a2, a3 = sts[0]
    for st in sts[1:]:
        for v in st:
            a0, a1, a2, a3 = _insert((a0, a1, a2, a3), v)

    ii = lax.broadcasted_iota(jnp.int32, (8, 128), 1)
    dcs = dc_ref[0]
    d2 = dcs * dcs
    cws = [1.0 / (1.0 + jnp.exp(jnp.float32(1.0))),
           1.0 / (1.0 + jnp.exp(dcs)),
           1.0 / (1.0 + jnp.exp(d2)),
           1.0 / (1.0 + jnp.exp(d2 * dcs))]
    denom = jnp.float32(p) - (cws[0] + cws[1] + (cws[2] + cws[3]))

    corr = jnp.zeros((8, 1), jnp.float32)
    for k in range(4):
        g = jnp.max(a0, axis=1, keepdims=True)
        first = jnp.min(jnp.where(a0 == g, ii, 128), axis=1, keepdims=True)
        hit = ii == first
        a0 = jnp.where(hit, a1, a0)
        a1 = jnp.where(hit, a2, a1)
        a2 = jnp.where(hit, a3, a2)
        a3 = jnp.where(hit, neg_inf, a3)
        corr = corr + cws[k] * g

    o_ref[...] = (total - corr) / denom


_SC_ROWS = 256


def kernel(x, dc):
    b, c, h, w = x.shape
    nrows, p = b * c, h * w
    info = plsc.get_sparse_core_info()
    nc, ns = info.num_cores, info.num_subcores
    nw = nc * ns
    assert _SC_ROWS % (8 * nw) == 0 and p % (128 * _U) == 0
    rpw = _SC_ROWS // nw

    xr = x.reshape(nrows, p)
    dcf = dc.astype(jnp.float32)
    dc16 = jnp.broadcast_to(dcf, (_L,))

    mesh = plsc.VectorSubcoreMesh(core_axis_name="c", subcore_axis_name="s")
    sc_kern = functools.partial(
        pl.kernel,
        mesh=mesh,
        out_type=jax.ShapeDtypeStruct((_SC_ROWS,), jnp.float32),
        scratch_types=[
            pltpu.VMEM((p,), jnp.float32),
            pltpu.VMEM((p,), jnp.float32),
            pltpu.VMEM((2 * _L,), jnp.float32),
            pltpu.VMEM((_L,), jnp.float32),
            pltpu.SemaphoreType.DMA,
            pltpu.SemaphoreType.DMA,
        ],
    )(functools.partial(_sc_body, nrows, p, rpw, nc))
    y_sc = sc_kern(xr, dc16)

    ntc = nrows - _SC_ROWS
    y_tc = pl.pallas_call(
        _tc_body,
        grid=(ntc // 8,),
        in_specs=[
            pl.BlockSpec(memory_space=pltpu.SMEM),
            pl.BlockSpec((8, p), lambda i: (i + _SC_ROWS // 8, 0)),
        ],
        out_specs=pl.BlockSpec((8, 1), lambda i: (i, 0)),
        out_shape=jax.ShapeDtypeStruct((ntc, 1), jnp.float32),
    )(dcf, xr)

    return jnp.concatenate([y_sc, y_tc[:, 0]]).reshape(b, c)
